# baseline (speedup 1.0000x reference)
import functools

import jax
import jax.numpy as jnp
from jax.experimental import pallas as pl
from jax.experimental.pallas import tpu as pltpu


def _moe_body(x_ref, gw_ref, w13_ref, w2_ref, out_ref, wmat_ref):
    e = pl.program_id(1)
    f = pl.program_id(2)
    x = x_ref[...]

    @pl.when((e == 0) & (f == 0))
    def _router():
        logits = jax.lax.dot_general(
            x, gw_ref[...], (((1,), (1,)), ((), ())),
            preferred_element_type=jnp.float32,
        )
        p = jax.nn.softmax(logits, axis=-1)
        p1 = jnp.max(p, axis=-1)
        i1 = jnp.argmax(p, axis=-1)
        ecols = jax.lax.broadcasted_iota(jnp.int32, p.shape, 1)
        p_m = jnp.where(ecols == i1[:, None], -jnp.inf, p)
        p2 = jnp.max(p_m, axis=-1)
        i2 = jnp.argmax(p_m, axis=-1)
        wmat = jnp.where(ecols == i1[:, None], p1[:, None], 0.0) + jnp.where(
            ecols == i2[:, None], p2[:, None], 0.0)
        wmat_ref[...] = wmat / (p1 + p2)[:, None]

    wmat = wmat_ref[...]
    cols = jax.lax.broadcasted_iota(jnp.int32, wmat.shape, 1)
    w_tok = jnp.sum(jnp.where(cols == e, wmat, 0.0), axis=1)

    g = jax.lax.dot_general(
        x, w13_ref[0, 0], (((1,), (1,)), ((), ())),
        preferred_element_type=jnp.float32,
    )
    u = jax.lax.dot_general(
        x, w13_ref[0, 1], (((1,), (1,)), ((), ())),
        preferred_element_type=jnp.float32,
    )
    h = g * jax.nn.sigmoid(g) * u
    y = jax.lax.dot_general(
        h, w2_ref[0], (((1,), (1,)), ((), ())),
        preferred_element_type=jnp.float32,
    )
    contrib = w_tok[:, None] * y

    @pl.when((e == 0) & (f == 0))
    def _init():
        out_ref[...] = contrib

    @pl.when((e != 0) | (f != 0))
    def _acc():
        out_ref[...] += contrib


def kernel(hidden_states, gate_weight, w13, w2):
    B, S, H = hidden_states.shape
    E, DFF2, _ = w13.shape
    dff = DFF2 // 2
    T = B * S
    TM = 1024
    FC = 256
    x = hidden_states.reshape(T, H)
    w13r = w13.reshape(E, 2, dff, H)

    out = pl.pallas_call(
        _moe_body,
        grid=(T // TM, E, dff // FC),
        in_specs=[
            pl.BlockSpec((TM, H), lambda i, e, f: (i, 0)),
            pl.BlockSpec((E, H), lambda i, e, f: (0, 0)),
            pl.BlockSpec((1, 2, FC, H), lambda i, e, f: (e, 0, f, 0)),
            pl.BlockSpec((1, H, FC), lambda i, e, f: (e, 0, f)),
        ],
        out_specs=pl.BlockSpec((TM, H), lambda i, e, f: (i, 0)),
        out_shape=jax.ShapeDtypeStruct((T, H), jnp.float32),
        scratch_shapes=[pltpu.VMEM((TM, E), jnp.float32)],
        compiler_params=pltpu.CompilerParams(
            dimension_semantics=("parallel", "arbitrary", "arbitrary"),
        ),
    )(x, gate_weight, w13r, w2)
    return out.reshape(B, S, H)


# speedup vs baseline: 1.1930x; 1.1930x over previous
import functools

import jax
import jax.numpy as jnp
from jax import lax
from jax.experimental import pallas as pl
from jax.experimental.pallas import tpu as pltpu
from jax.experimental.pallas import tpu_sc as plsc

TM_R = 1024
TM_G = 256


def _router_body(x_ref, gw_ref, w_ref, i_ref):
    x = x_ref[...]
    logits = lax.dot_general(
        x, gw_ref[...], (((1,), (1,)), ((), ())),
        preferred_element_type=jnp.float32,
    )
    p = jax.nn.softmax(logits, axis=-1)
    p1 = jnp.max(p, axis=-1)
    i1 = jnp.argmax(p, axis=-1)
    ecols = lax.broadcasted_iota(jnp.int32, p.shape, 1)
    p_m = jnp.where(ecols == i1[:, None], -jnp.inf, p)
    p2 = jnp.max(p_m, axis=-1)
    i2 = jnp.argmax(p_m, axis=-1)
    s = p1 + p2
    w_ref[0, :] = p1 / s
    w_ref[1, :] = p2 / s
    i_ref[0, :] = i1.astype(jnp.int32)
    i_ref[1, :] = i2.astype(jnp.int32)


def _router(x, gw, T, E, H):
    return pl.pallas_call(
        _router_body,
        grid=(T // TM_R,),
        in_specs=[
            pl.BlockSpec((TM_R, H), lambda i: (i, 0)),
            pl.BlockSpec((E, H), lambda i: (0, 0)),
        ],
        out_specs=[
            pl.BlockSpec((2, TM_R), lambda i: (0, i)),
            pl.BlockSpec((2, TM_R), lambda i: (0, i)),
        ],
        out_shape=[
            jax.ShapeDtypeStruct((2, T), jnp.float32),
            jax.ShapeDtypeStruct((2, T), jnp.int32),
        ],
    )(x, gw)


def _make_sc_gather(T, H, P):
    info = plsc.get_sparse_core_info()
    NC, NS = info.num_cores, info.num_subcores
    NW = NC * NS
    rpw = P // NW
    CH = 32
    n_ch = rpw // CH
    mesh = plsc.VectorSubcoreMesh(core_axis_name="c", subcore_axis_name="s")

    @functools.partial(
        pl.kernel, mesh=mesh,
        out_type=jax.ShapeDtypeStruct((P, H), jnp.float32),
        scratch_types=[
            pltpu.VMEM((rpw,), jnp.int32),
            pltpu.VMEM((CH, H), jnp.float32),
            pltpu.SemaphoreType.DMA,
        ],
    )
    def gather_k(x_hbm, idx_hbm, out_hbm, idx_v, rows_v, sem):
        wid = lax.axis_index("s") * NC + lax.axis_index("c")
        base = wid * rpw
        pltpu.sync_copy(idx_hbm.at[pl.ds(base, rpw)], idx_v)

        def chunk(c, _):
            pltpu.async_copy(
                x_hbm.at[idx_v.at[pl.ds(c * CH, CH)]], rows_v, sem).wait()
            pltpu.sync_copy(rows_v, out_hbm.at[pl.ds(base + c * CH, CH)])
            return ()

        lax.fori_loop(0, n_ch, chunk, ())

    return gather_k


def _make_sc_combine(T, H, P):
    info = plsc.get_sparse_core_info()
    NC, NS = info.num_cores, info.num_subcores
    NW = NC * NS
    tpw = T // NW
    CH = 16
    n_ch = tpw // CH
    mesh = plsc.VectorSubcoreMesh(core_axis_name="c", subcore_axis_name="s")

    @functools.partial(
        pl.kernel, mesh=mesh,
        out_type=jax.ShapeDtypeStruct((T, H), jnp.float32),
        scratch_types=[
            pltpu.VMEM((tpw,), jnp.int32),
            pltpu.VMEM((tpw,), jnp.int32),
            pltpu.VMEM((CH, H), jnp.float32),
            pltpu.VMEM((CH, H), jnp.float32),
            pltpu.SemaphoreType.DMA,
            pltpu.SemaphoreType.DMA,
        ],
    )
    def combine_k(y_hbm, p0_hbm, p1_hbm, out_hbm, p0_v, p1_v, buf_a, buf_b,
                  sem_a, sem_b):
        wid = lax.axis_index("s") * NC + lax.axis_index("c")
        base = wid * tpw
        pltpu.sync_copy(p0_hbm.at[pl.ds(base, tpw)], p0_v)
        pltpu.sync_copy(p1_hbm.at[pl.ds(base, tpw)], p1_v)

        def chunk(c, _):
            cp_a = pltpu.async_copy(
                y_hbm.at[p0_v.at[pl.ds(c * CH, CH)]], buf_a, sem_a)
            cp_b = pltpu.async_copy(
                y_hbm.at[p1_v.at[pl.ds(c * CH, CH)]], buf_b, sem_b)
            cp_a.wait()
            cp_b.wait()

            def row(r, _):
                def col(j, _):
                    o = pl.multiple_of(j * 16, 16)
                    buf_a[r, pl.ds(o, 16)] = (
                        buf_a[r, pl.ds(o, 16)] + buf_b[r, pl.ds(o, 16)])
                    return ()
                lax.fori_loop(0, H // 16, col, (), unroll=8)
                return ()

            lax.fori_loop(0, CH, row, ())
            pltpu.sync_copy(buf_a, out_hbm.at[pl.ds(base + c * CH, CH)])
            return ()

        lax.fori_loop(0, n_ch, chunk, ())

    return combine_k


def _ffn_body(te_ref, x_ref, w13_ref, w2_ref, wp_ref, y_ref, *, dff):
    x = x_ref[...].astype(jnp.bfloat16)
    g = lax.dot_general(
        x, w13_ref[0, 0], (((1,), (1,)), ((), ())),
        preferred_element_type=jnp.float32,
    )
    u = lax.dot_general(
        x, w13_ref[0, 1], (((1,), (1,)), ((), ())),
        preferred_element_type=jnp.float32,
    )
    h = (g * jax.nn.sigmoid(g) * u).astype(jnp.bfloat16)
    y = lax.dot_general(
        h, w2_ref[0], (((1,), (1,)), ((), ())),
        preferred_element_type=jnp.float32,
    )
    y_ref[...] = wp_ref[0, 0][:, None] * y


def _grouped_ffn(x_sorted, w13r, w2, w_pad, tile_expert, P, H, dff, n_tiles):
    grid_spec = pltpu.PrefetchScalarGridSpec(
        num_scalar_prefetch=1,
        grid=(n_tiles,),
        in_specs=[
            pl.BlockSpec((TM_G, H), lambda i, te: (i, 0)),
            pl.BlockSpec((1, 2, dff, H), lambda i, te: (te[i], 0, 0, 0)),
            pl.BlockSpec((1, H, dff), lambda i, te: (te[i], 0, 0)),
            pl.BlockSpec((1, 1, TM_G), lambda i, te: (i, 0, 0)),
        ],
        out_specs=pl.BlockSpec((TM_G, H), lambda i, te: (i, 0)),
    )
    return pl.pallas_call(
        functools.partial(_ffn_body, dff=dff),
        grid_spec=grid_spec,
        out_shape=jax.ShapeDtypeStruct((P, H), jnp.float32),
    )(tile_expert, x_sorted, w13r, w2, w_pad)


def kernel(hidden_states, gate_weight, w13, w2):
    B, S, H = hidden_states.shape
    E, DFF2, _ = w13.shape
    dff = DFF2 // 2
    T = B * S
    TK = 2 * T
    P = TK + E * TM_G
    n_tiles = P // TM_G

    x = hidden_states.reshape(T, H)

    topw, topi = _router(x, gate_weight, T, E, H)

    flat_e = jnp.concatenate([topi[0], topi[1]])
    order = jnp.argsort(flat_e, stable=True).astype(jnp.int32)
    e_sorted = flat_e[order]
    onehot = (flat_e[:, None] == jnp.arange(E, dtype=jnp.int32)[None, :])
    counts = jnp.sum(onehot.astype(jnp.int32), axis=0)
    starts = jnp.concatenate(
        [jnp.zeros((1,), jnp.int32), jnp.cumsum(counts)[:-1]])
    padded = ((counts + TM_G - 1) // TM_G) * TM_G
    pad_off = jnp.concatenate(
        [jnp.zeros((1,), jnp.int32), jnp.cumsum(padded)[:-1]])
    jj = jnp.arange(TK, dtype=jnp.int32)
    dest = pad_off[e_sorted] + jj - starts[e_sorted]
    row_id = jnp.zeros((P,), jnp.int32).at[dest].set(order % T)
    pos = jnp.zeros((TK,), jnp.int32).at[order].set(dest)
    pos0, pos1 = pos[:T], pos[T:]
    w_flat = jnp.concatenate([topw[0], topw[1]])
    w_pad = jnp.zeros((P,), jnp.float32).at[dest].set(w_flat[order])
    w_pad = w_pad.reshape(n_tiles, 1, TM_G)
    pad_end = jnp.cumsum(padded)
    tile_expert = jnp.searchsorted(
        pad_end, jnp.arange(n_tiles, dtype=jnp.int32) * TM_G, side="right")
    tile_expert = jnp.minimum(tile_expert, E - 1).astype(jnp.int32)

    x_sorted = _make_sc_gather(T, H, P)(x, row_id)

    w13r = w13.reshape(E, 2, dff, H).astype(jnp.bfloat16)
    w2b = w2.astype(jnp.bfloat16)
    y_sorted = _grouped_ffn(x_sorted, w13r, w2b, w_pad, tile_expert,
                            P, H, dff, n_tiles)

    out = _make_sc_combine(T, H, P)(y_sorted, pos0, pos1)
    return out.reshape(B, S, H)


# speedup vs baseline: 1.4371x; 1.2047x over previous
import functools

import jax
import jax.numpy as jnp
from jax import lax
from jax.experimental import pallas as pl
from jax.experimental.pallas import tpu as pltpu
from jax.experimental.pallas import tpu_sc as plsc

TM_R = 1024
TM_G = 256


def _router_body(x_ref, gw_ref, w_ref, i_ref):
    x = x_ref[...]
    logits = lax.dot_general(
        x, gw_ref[...], (((1,), (1,)), ((), ())),
        preferred_element_type=jnp.float32,
    )
    p = jax.nn.softmax(logits, axis=-1)
    p1 = jnp.max(p, axis=-1)
    i1 = jnp.argmax(p, axis=-1)
    ecols = lax.broadcasted_iota(jnp.int32, p.shape, 1)
    p_m = jnp.where(ecols == i1[:, None], -jnp.inf, p)
    p2 = jnp.max(p_m, axis=-1)
    i2 = jnp.argmax(p_m, axis=-1)
    s = p1 + p2
    w_ref[0, :] = p1 / s
    w_ref[1, :] = p2 / s
    i_ref[0, :] = i1.astype(jnp.int32)
    i_ref[1, :] = i2.astype(jnp.int32)


def _router(x, gw, T, E, H):
    return pl.pallas_call(
        _router_body,
        grid=(T // TM_R,),
        in_specs=[
            pl.BlockSpec((TM_R, H), lambda i: (i, 0)),
            pl.BlockSpec((E, H), lambda i: (0, 0)),
        ],
        out_specs=[
            pl.BlockSpec((2, TM_R), lambda i: (0, i)),
            pl.BlockSpec((2, TM_R), lambda i: (0, i)),
        ],
        out_shape=[
            jax.ShapeDtypeStruct((2, T), jnp.float32),
            jax.ShapeDtypeStruct((2, T), jnp.int32),
        ],
    )(x, gw)


def _make_sc_gather(T, H, P):
    info = plsc.get_sparse_core_info()
    NC, NS = info.num_cores, info.num_subcores
    NW = NC * NS
    rpw = P // NW
    CH = 24
    n_ch = rpw // CH
    assert rpw % CH == 0 and n_ch % 2 == 0
    mesh = plsc.VectorSubcoreMesh(core_axis_name="c", subcore_axis_name="s")

    @functools.partial(
        pl.kernel, mesh=mesh,
        out_type=jax.ShapeDtypeStruct((P, H), jnp.float32),
        scratch_types=[
            pltpu.VMEM((rpw,), jnp.int32),
            pltpu.VMEM((CH, H), jnp.float32),
            pltpu.VMEM((CH, H), jnp.float32),
            pltpu.SemaphoreType.DMA,
            pltpu.SemaphoreType.DMA,
        ],
    )
    def gather_k(x_hbm, idx_hbm, out_hbm, idx_v, buf0, buf1, sem0, sem1):
        wid = lax.axis_index("s") * NC + lax.axis_index("c")
        base = wid * rpw
        pltpu.sync_copy(idx_hbm.at[pl.ds(base, rpw)], idx_v)

        def start(c, buf, sem):
            pltpu.async_copy(
                x_hbm.at[idx_v.at[pl.ds(c * CH, CH)]], buf, sem)

        def drain(c, buf, sem):
            pltpu.make_async_copy(
                x_hbm.at[idx_v.at[pl.ds(c * CH, CH)]], buf, sem).wait()
            pltpu.sync_copy(buf, out_hbm.at[pl.ds(base + c * CH, CH)])

        start(0, buf0, sem0)

        def pair(c2, _):
            c0 = c2 * 2
            start(c0 + 1, buf1, sem1)
            drain(c0, buf0, sem0)

            @pl.when(c0 + 2 < n_ch)
            def _():
                start(c0 + 2, buf0, sem0)

            drain(c0 + 1, buf1, sem1)
            return ()

        lax.fori_loop(0, n_ch // 2, pair, ())

    return gather_k


def _make_sc_combine(T, H, P):
    info = plsc.get_sparse_core_info()
    NC, NS = info.num_cores, info.num_subcores
    NW = NC * NS
    tpw = T // NW
    CH = 8
    n_ch = tpw // CH
    assert tpw % CH == 0 and n_ch % 2 == 0
    mesh = plsc.VectorSubcoreMesh(core_axis_name="c", subcore_axis_name="s")

    @functools.partial(
        pl.kernel, mesh=mesh,
        out_type=jax.ShapeDtypeStruct((T, H), jnp.float32),
        scratch_types=[
            pltpu.VMEM((tpw,), jnp.int32),
            pltpu.VMEM((tpw,), jnp.int32),
            pltpu.VMEM((CH, H), jnp.float32),
            pltpu.VMEM((CH, H), jnp.float32),
            pltpu.VMEM((CH, H), jnp.float32),
            pltpu.VMEM((CH, H), jnp.float32),
            pltpu.SemaphoreType.DMA,
            pltpu.SemaphoreType.DMA,
        ],
    )
    def combine_k(y_hbm, p0_hbm, p1_hbm, out_hbm, p0_v, p1_v,
                  buf_a0, buf_b0, buf_a1, buf_b1, sem0, sem1):
        wid = lax.axis_index("s") * NC + lax.axis_index("c")
        base = wid * tpw
        pltpu.sync_copy(p0_hbm.at[pl.ds(base, tpw)], p0_v)
        pltpu.sync_copy(p1_hbm.at[pl.ds(base, tpw)], p1_v)

        def start(c, buf_a, buf_b, sem):
            pltpu.async_copy(y_hbm.at[p0_v.at[pl.ds(c * CH, CH)]], buf_a, sem)
            pltpu.async_copy(y_hbm.at[p1_v.at[pl.ds(c * CH, CH)]], buf_b, sem)

        def drain(c, buf_a, buf_b, sem):
            pltpu.make_async_copy(
                y_hbm.at[p0_v.at[pl.ds(c * CH, CH)]], buf_a, sem).wait()
            pltpu.make_async_copy(
                y_hbm.at[p1_v.at[pl.ds(c * CH, CH)]], buf_b, sem).wait()

            def row(r, _):
                def col(j, _):
                    o = pl.multiple_of(j * 16, 16)
                    buf_a[r, pl.ds(o, 16)] = (
                        buf_a[r, pl.ds(o, 16)] + buf_b[r, pl.ds(o, 16)])
                    return ()
                lax.fori_loop(0, H // 16, col, (), unroll=8)
                return ()

            lax.fori_loop(0, CH, row, ())
            pltpu.sync_copy(buf_a, out_hbm.at[pl.ds(base + c * CH, CH)])

        start(0, buf_a0, buf_b0, sem0)

        def pair(c2, _):
            c0 = c2 * 2
            start(c0 + 1, buf_a1, buf_b1, sem1)
            drain(c0, buf_a0, buf_b0, sem0)

            @pl.when(c0 + 2 < n_ch)
            def _():
                start(c0 + 2, buf_a0, buf_b0, sem0)

            drain(c0 + 1, buf_a1, buf_b1, sem1)
            return ()

        lax.fori_loop(0, n_ch // 2, pair, ())

    return combine_k


def _ffn_body(te_ref, x_ref, w13_ref, w2_ref, wp_ref, y_ref, *, dff):
    x = x_ref[...]
    g = lax.dot_general(
        x, w13_ref[0, 0], (((1,), (1,)), ((), ())),
        preferred_element_type=jnp.float32,
    )
    u = lax.dot_general(
        x, w13_ref[0, 1], (((1,), (1,)), ((), ())),
        preferred_element_type=jnp.float32,
    )
    h = g * jax.nn.sigmoid(g) * u
    y = lax.dot_general(
        h, w2_ref[0], (((1,), (1,)), ((), ())),
        preferred_element_type=jnp.float32,
    )
    y_ref[...] = wp_ref[0, 0][:, None] * y


def _grouped_ffn(x_sorted, w13r, w2, w_pad, tile_expert, P, H, dff, n_tiles):
    grid_spec = pltpu.PrefetchScalarGridSpec(
        num_scalar_prefetch=1,
        grid=(n_tiles,),
        in_specs=[
            pl.BlockSpec((TM_G, H), lambda i, te: (i, 0)),
            pl.BlockSpec((1, 2, dff, H), lambda i, te: (te[i], 0, 0, 0)),
            pl.BlockSpec((1, H, dff), lambda i, te: (te[i], 0, 0)),
            pl.BlockSpec((1, 1, TM_G), lambda i, te: (i, 0, 0)),
        ],
        out_specs=pl.BlockSpec((TM_G, H), lambda i, te: (i, 0)),
    )
    return pl.pallas_call(
        functools.partial(_ffn_body, dff=dff),
        grid_spec=grid_spec,
        out_shape=jax.ShapeDtypeStruct((P, H), jnp.float32),
    )(tile_expert, x_sorted, w13r, w2, w_pad)


def kernel(hidden_states, gate_weight, w13, w2):
    B, S, H = hidden_states.shape
    E, DFF2, _ = w13.shape
    dff = DFF2 // 2
    T = B * S
    TK = 2 * T
    P = TK + E * TM_G
    n_tiles = P // TM_G

    x = hidden_states.reshape(T, H)

    topw, topi = _router(x, gate_weight, T, E, H)

    flat_e = jnp.concatenate([topi[0], topi[1]])
    order = jnp.argsort(flat_e, stable=True).astype(jnp.int32)
    e_sorted = flat_e[order]
    onehot = (flat_e[:, None] == jnp.arange(E, dtype=jnp.int32)[None, :])
    counts = jnp.sum(onehot.astype(jnp.int32), axis=0)
    starts = jnp.concatenate(
        [jnp.zeros((1,), jnp.int32), jnp.cumsum(counts)[:-1]])
    padded = ((counts + TM_G - 1) // TM_G) * TM_G
    pad_off = jnp.concatenate(
        [jnp.zeros((1,), jnp.int32), jnp.cumsum(padded)[:-1]])
    jj = jnp.arange(TK, dtype=jnp.int32)
    dest = pad_off[e_sorted] + jj - starts[e_sorted]
    row_id = jnp.zeros((P,), jnp.int32).at[dest].set(order % T)
    pos = jnp.zeros((TK,), jnp.int32).at[order].set(dest)
    pos0, pos1 = pos[:T], pos[T:]
    w_flat = jnp.concatenate([topw[0], topw[1]])
    w_pad = jnp.zeros((P,), jnp.float32).at[dest].set(w_flat[order])
    w_pad = w_pad.reshape(n_tiles, 1, TM_G)
    pad_end = jnp.cumsum(padded)
    tile_expert = jnp.searchsorted(
        pad_end, jnp.arange(n_tiles, dtype=jnp.int32) * TM_G, side="right")
    tile_expert = jnp.minimum(tile_expert, E - 1).astype(jnp.int32)

    x_sorted = _make_sc_gather(T, H, P)(x, row_id)

    w13r = w13.reshape(E, 2, dff, H)
    y_sorted = _grouped_ffn(x_sorted, w13r, w2, w_pad, tile_expert,
                            P, H, dff, n_tiles)

    out = _make_sc_combine(T, H, P)(y_sorted, pos0, pos1)
    return out.reshape(B, S, H)
